# Initial kernel scaffold; baseline (speedup 1.0000x reference)
import jax
import jax.numpy as jnp
from jax.experimental import pallas as pl


def kernel(h, neighbor_idx, neighbor_mask, Wq, Wk, Wv, Wfc, bfc, gamma, beta):
    raise NotImplementedError("write your pallas kernel here")


# speedup vs baseline: 1.8755x; 1.8755x over previous
import functools
import math

import jax
import jax.numpy as jnp
from jax import lax
from jax.experimental import pallas as pl
from jax.experimental.pallas import tpu as pltpu
from jax.experimental.pallas import tpu_sc as plsc

N = 10000
D = 128
OUT_DIM = 128
NHEAD = 4
HEAD_DIM = 32
DEG = 32

NW = 32
NP_W = 320
NPAD = NW * NP_W
OB = 64
N_OB = NP_W // OB
CH = 4
N_CH = OB // CH

_INV_SQRT_HD = 1.0 / math.sqrt(HEAD_DIM)


def _proj_body(h_ref, wq_ref, wk_ref, wv_ref, q_ref, kv_ref):
    hb = h_ref[...]
    dn = (((1,), (1,)), ((), ()))
    q = lax.dot_general(hb, wq_ref[...], dn, preferred_element_type=jnp.float32)
    k = lax.dot_general(hb, wk_ref[...], dn, preferred_element_type=jnp.float32)
    v = lax.dot_general(hb, wv_ref[...], dn, preferred_element_type=jnp.float32)
    q_ref[...] = q
    kv_ref[...] = jnp.concatenate([k, v], axis=1)


def _project(h_p, Wq, Wk, Wv):
    bs = 1024
    grid = (NPAD // bs,)
    return pl.pallas_call(
        _proj_body,
        grid=grid,
        in_specs=[
            pl.BlockSpec((bs, D), lambda i: (i, 0)),
            pl.BlockSpec((OUT_DIM, D), lambda i: (0, 0)),
            pl.BlockSpec((OUT_DIM, D), lambda i: (0, 0)),
            pl.BlockSpec((OUT_DIM, D), lambda i: (0, 0)),
        ],
        out_specs=[
            pl.BlockSpec((bs, OUT_DIM), lambda i: (i, 0)),
            pl.BlockSpec((bs, 2 * OUT_DIM), lambda i: (i, 0)),
        ],
        out_shape=[
            jax.ShapeDtypeStruct((NPAD, OUT_DIM), jnp.float32),
            jax.ShapeDtypeStruct((NPAD, 2 * OUT_DIM), jnp.float32),
        ],
    )(h_p, Wq, Wk, Wv)


def _node_compute(kv_buf, q_buf, mask_buf, alpha_buf, out_buf, iota16, node, nl):
    rows0 = iota16 + (nl * DEG)
    rows1 = rows0 + 16
    zinv = []
    for h in range(NHEAD):
        def sbody(i, acc, h=h):
            a0, a1 = acc
            base = h * HEAD_DIM + i * 16
            qv = q_buf[node, pl.ds(base, 16)]
            colb = jnp.full((16,), base, jnp.int32)
            for k in range(16):
                qs = qv[k]
                col = colb + k
                g0 = plsc.load_gather(kv_buf, [rows0, col])
                g1 = plsc.load_gather(kv_buf, [rows1, col])
                a0 = a0 + qs * g0
                a1 = a1 + qs * g1
            return (a0, a1)
        s0, s1 = lax.fori_loop(0, 2, sbody,
                               (jnp.zeros((16,), jnp.float32),
                                jnp.zeros((16,), jnp.float32)))
        s0 = s0 * _INV_SQRT_HD
        s1 = s1 * _INV_SQRT_HD
        m0 = mask_buf[node, pl.ds(0, 16)]
        m1 = mask_buf[node, pl.ds(16, 16)]
        s0 = jnp.where(m0 == 0.0, -1e9, s0)
        s1 = jnp.where(m1 == 0.0, -1e9, s1)
        mx = jnp.max(jnp.maximum(s0, s1))
        e0 = jnp.exp(s0 - mx)
        e1 = jnp.exp(s1 - mx)
        z = jnp.sum(e0 + e1)
        alpha_buf[pl.ds(h * DEG, 16)] = e0
        alpha_buf[pl.ds(h * DEG + 16, 16)] = e1
        zinv.append(1.0 / jnp.full((16,), z, jnp.float32))

    def wbody(g, accs):
        av = [alpha_buf[pl.ds(h * DEG + g * 16, 16)] for h in range(NHEAD)]
        accs = list(accs)
        for j in range(16):
            row = nl * DEG + g * 16 + j
            for k in range(8):
                vk = kv_buf[row, pl.ds(OUT_DIM + 16 * k, 16)]
                accs[k] = accs[k] + av[k // 2][j] * vk
        return tuple(accs)

    accs = lax.fori_loop(0, 2, wbody,
                         tuple(jnp.zeros((16,), jnp.float32) for _ in range(8)))
    for k in range(8):
        out_buf[node, pl.ds(16 * k, 16)] = accs[k] * zinv[k // 2]


def _sc_body(kv_hbm, q_hbm, idx_hbm, mask_hbm, out_hbm,
             kv_buf, q_buf, idx_buf, mask_buf, out_buf, alpha_buf, sem):
    cid = lax.axis_index("c")
    sid = lax.axis_index("s")
    wid = sid * 2 + cid
    iota16 = lax.iota(jnp.int32, 16)

    def ob_body(ob, _):
        node0 = wid * NP_W + ob * OB
        pltpu.sync_copy(q_hbm.at[pl.ds(node0, OB)], q_buf)
        pltpu.sync_copy(idx_hbm.at[pl.ds(wid * (NP_W * DEG // 128) + ob * N_CH, N_CH)],
                        idx_buf)
        pltpu.sync_copy(mask_hbm.at[pl.ds(node0, OB)], mask_buf)

        def c_body(c, _):
            pltpu.async_copy(kv_hbm.at[idx_buf.at[c]], kv_buf, sem).wait()
            for nl in range(CH):
                _node_compute(kv_buf, q_buf, mask_buf, alpha_buf, out_buf,
                              iota16, c * CH + nl, nl)
            return 0

        lax.fori_loop(0, N_CH, c_body, 0)
        pltpu.sync_copy(out_buf, out_hbm.at[pl.ds(node0, OB)])
        return 0

    lax.fori_loop(0, N_OB, ob_body, 0)


def _sc_attention(kv, q, idx2d, mask_p):
    mesh = plsc.VectorSubcoreMesh(core_axis_name="c", subcore_axis_name="s")
    f = functools.partial(
        pl.kernel,
        mesh=mesh,
        compiler_params=pltpu.CompilerParams(use_tc_tiling_on_sc=False,
                                             needs_layout_passes=False),
        out_type=jax.ShapeDtypeStruct((NPAD, OUT_DIM), jnp.float32),
        scratch_types=[
            pltpu.VMEM((CH * DEG, 2 * OUT_DIM), jnp.float32),
            pltpu.VMEM((OB, OUT_DIM), jnp.float32),
            pltpu.VMEM((OB * DEG // 128, 128), jnp.int32),
            pltpu.VMEM((OB, DEG), jnp.float32),
            pltpu.VMEM((OB, OUT_DIM), jnp.float32),
            pltpu.VMEM((NHEAD * DEG,), jnp.float32),
            pltpu.SemaphoreType.DMA,
        ],
    )(_sc_body)
    return f(kv, q, idx2d, mask_p)


def _post_body(hd_ref, q_ref, wfc_ref, bfc_ref, gamma_ref, beta_ref, o_ref):
    dn = (((1,), (1,)), ((), ()))
    x = lax.dot_general(hd_ref[...], wfc_ref[...], dn,
                        preferred_element_type=jnp.float32)
    x = x + bfc_ref[...] + q_ref[...]
    x = 0.5 * x * (1.0 + lax.erf(x * (1.0 / math.sqrt(2.0))))
    mean = jnp.mean(x, axis=1, keepdims=True)
    xc = x - mean
    var = jnp.mean(xc * xc, axis=1, keepdims=True)
    o_ref[...] = xc * lax.rsqrt(var + 1e-5) * gamma_ref[...] + beta_ref[...]


def _postprocess(heads, q, Wfc, bfc, gamma, beta):
    bs = 1000
    grid = (N // bs,)
    return pl.pallas_call(
        _post_body,
        grid=grid,
        in_specs=[
            pl.BlockSpec((bs, OUT_DIM), lambda i: (i, 0)),
            pl.BlockSpec((bs, OUT_DIM), lambda i: (i, 0)),
            pl.BlockSpec((OUT_DIM, OUT_DIM), lambda i: (0, 0)),
            pl.BlockSpec((1, OUT_DIM), lambda i: (0, 0)),
            pl.BlockSpec((1, OUT_DIM), lambda i: (0, 0)),
            pl.BlockSpec((1, OUT_DIM), lambda i: (0, 0)),
        ],
        out_specs=pl.BlockSpec((bs, OUT_DIM), lambda i: (i, 0)),
        out_shape=jax.ShapeDtypeStruct((N, OUT_DIM), jnp.float32),
    )(heads, q, Wfc, bfc.reshape(1, OUT_DIM), gamma.reshape(1, OUT_DIM),
      beta.reshape(1, OUT_DIM))


def kernel(h, neighbor_idx, neighbor_mask, Wq, Wk, Wv, Wfc, bfc, gamma, beta):
    h_p = jnp.pad(h, ((0, NPAD - N), (0, 0)))
    q, kv = _project(h_p, Wq, Wk, Wv)
    idx_p = jnp.pad(neighbor_idx.astype(jnp.int32), ((0, NPAD - N), (0, 0)))
    mask_p = jnp.pad(neighbor_mask, ((0, NPAD - N), (0, 0)), constant_values=1.0)
    idx2d = idx_p.reshape(NPAD * DEG // 128, 128)
    heads = _sc_attention(kv, q, idx2d, mask_p)
    return _postprocess(heads[:N], q[:N], Wfc, bfc, gamma, beta)


# speedup vs baseline: 1.9827x; 1.0571x over previous
import functools
import math

import jax
import jax.numpy as jnp
from jax import lax
from jax.experimental import pallas as pl
from jax.experimental.pallas import tpu as pltpu
from jax.experimental.pallas import tpu_sc as plsc

N = 10000
D = 128
OUT_DIM = 128
NHEAD = 4
HEAD_DIM = 32
DEG = 32

NW = 32
NP_W = 320
NPAD = NW * NP_W
OB = 64
N_OB = NP_W // OB
CH = 4
N_CH = OB // CH

_INV_SQRT_HD = 1.0 / math.sqrt(HEAD_DIM)
KVW = 256


def _proj_body(h_ref, wq_ref, wk_ref, wv_ref, q_ref, kv_ref):
    hb = h_ref[...]
    dn = (((1,), (1,)), ((), ()))
    q = lax.dot_general(hb, wq_ref[...], dn, preferred_element_type=jnp.float32)
    k = lax.dot_general(hb, wk_ref[...], dn, preferred_element_type=jnp.float32)
    v = lax.dot_general(hb, wv_ref[...], dn, preferred_element_type=jnp.float32)
    q_ref[...] = q
    kv_ref[:, 0:OUT_DIM] = k
    kv_ref[:, OUT_DIM:2 * OUT_DIM] = v


def _project(h_p, Wq, Wk, Wv):
    bs = 1024
    grid = (NPAD // bs,)
    return pl.pallas_call(
        _proj_body,
        grid=grid,
        in_specs=[
            pl.BlockSpec((bs, D), lambda i: (i, 0)),
            pl.BlockSpec((OUT_DIM, D), lambda i: (0, 0)),
            pl.BlockSpec((OUT_DIM, D), lambda i: (0, 0)),
            pl.BlockSpec((OUT_DIM, D), lambda i: (0, 0)),
        ],
        out_specs=[
            pl.BlockSpec((bs, OUT_DIM), lambda i: (i, 0)),
            pl.BlockSpec((bs, KVW), lambda i: (i, 0)),
        ],
        out_shape=[
            jax.ShapeDtypeStruct((NPAD, OUT_DIM), jnp.float32),
            jax.ShapeDtypeStruct((NPAD, KVW), jnp.float32),
        ],
    )(h_p, Wq, Wk, Wv)


def _node_compute(kv_buf, q_buf, mask_buf, alpha_buf, out_buf, iota16, node, nl):
    qv = [q_buf[node, pl.ds(16 * i, 16)] for i in range(8)]
    lane0 = iota16 == 0

    def sbody(j, _):
        row = nl * DEG + j
        for h in range(NHEAD):
            p = (qv[2 * h] * kv_buf[row, pl.ds(HEAD_DIM * h, 16)]
                 + qv[2 * h + 1] * kv_buf[row, pl.ds(HEAD_DIM * h + 16, 16)])
            s = jnp.sum(p) * _INV_SQRT_HD
            plsc.store_scatter(alpha_buf,
                               [jnp.full((16,), h * DEG + j, jnp.int32)],
                               jnp.full((16,), s, jnp.float32), mask=lane0)
        return 0

    lax.fori_loop(0, DEG, sbody, 0)

    zinv = []
    for h in range(NHEAD):
        s0 = alpha_buf[pl.ds(h * DEG, 16)]
        s1 = alpha_buf[pl.ds(h * DEG + 16, 16)]
        m0 = mask_buf[node, pl.ds(0, 16)]
        m1 = mask_buf[node, pl.ds(16, 16)]
        s0 = jnp.where(m0 == 0.0, -1e9, s0)
        s1 = jnp.where(m1 == 0.0, -1e9, s1)
        mx = jnp.max(jnp.maximum(s0, s1))
        e0 = jnp.exp(s0 - mx)
        e1 = jnp.exp(s1 - mx)
        z = jnp.sum(e0 + e1)
        alpha_buf[pl.ds(h * DEG, 16)] = e0
        alpha_buf[pl.ds(h * DEG + 16, 16)] = e1
        zinv.append(1.0 / jnp.full((16,), z, jnp.float32))

    def wbody(g, accs):
        av = [alpha_buf[pl.ds(h * DEG + g * 16, 16)] for h in range(NHEAD)]
        accs = list(accs)
        for j in range(16):
            row = nl * DEG + g * 16 + j
            for k in range(8):
                vk = kv_buf[row, pl.ds(OUT_DIM + 16 * k, 16)]
                accs[k] = accs[k] + av[k // 2][j] * vk
        return tuple(accs)

    accs = lax.fori_loop(0, 2, wbody,
                         tuple(jnp.zeros((16,), jnp.float32) for _ in range(8)))
    for k in range(8):
        out_buf[node, pl.ds(16 * k, 16)] = accs[k] * zinv[k // 2]


def _sc_body(kv_hbm, q_hbm, idx_hbm, mask_hbm, out_hbm,
             kv_buf0, kv_buf1, q_buf, idx_buf, mask_buf, out_buf, alpha_buf,
             sem0, sem1):
    cid = lax.axis_index("c")
    sid = lax.axis_index("s")
    wid = sid * 2 + cid
    iota16 = lax.iota(jnp.int32, 16)

    def ob_body(ob, _):
        node0 = wid * NP_W + ob * OB
        pltpu.sync_copy(q_hbm.at[pl.ds(node0, OB)], q_buf)
        pltpu.sync_copy(idx_hbm.at[pl.ds(wid * (NP_W * DEG // 128) + ob * N_CH, N_CH)],
                        idx_buf)
        pltpu.sync_copy(mask_hbm.at[pl.ds(node0, OB)], mask_buf)
        pltpu.async_copy(kv_hbm.at[idx_buf.at[0]], kv_buf0, sem0)

        def cc_body(cc, _):
            pltpu.async_copy(kv_hbm.at[idx_buf.at[2 * cc + 1]], kv_buf1, sem1)
            pltpu.make_async_copy(kv_hbm.at[idx_buf.at[0]], kv_buf0, sem0).wait()
            for nl in range(CH):
                _node_compute(kv_buf0, q_buf, mask_buf, alpha_buf, out_buf,
                              iota16, (2 * cc) * CH + nl, nl)

            @pl.when(cc < N_CH // 2 - 1)
            def _():
                pltpu.async_copy(kv_hbm.at[idx_buf.at[2 * cc + 2]], kv_buf0, sem0)

            pltpu.make_async_copy(kv_hbm.at[idx_buf.at[0]], kv_buf1, sem1).wait()
            for nl in range(CH):
                _node_compute(kv_buf1, q_buf, mask_buf, alpha_buf, out_buf,
                              iota16, (2 * cc + 1) * CH + nl, nl)
            return 0

        lax.fori_loop(0, N_CH // 2, cc_body, 0)
        pltpu.sync_copy(out_buf, out_hbm.at[pl.ds(node0, OB)])
        return 0

    lax.fori_loop(0, N_OB, ob_body, 0)


def _sc_attention(kv, q, idx2d, mask_p):
    mesh = plsc.VectorSubcoreMesh(core_axis_name="c", subcore_axis_name="s")
    f = functools.partial(
        pl.kernel,
        mesh=mesh,
        compiler_params=pltpu.CompilerParams(use_tc_tiling_on_sc=False,
                                             needs_layout_passes=False),
        out_type=jax.ShapeDtypeStruct((NPAD, OUT_DIM), jnp.float32),
        scratch_types=[
            pltpu.VMEM((CH * DEG, KVW), jnp.float32),
            pltpu.VMEM((CH * DEG, KVW), jnp.float32),
            pltpu.VMEM((OB, OUT_DIM), jnp.float32),
            pltpu.VMEM((OB * DEG // 128, 128), jnp.int32),
            pltpu.VMEM((OB, DEG), jnp.float32),
            pltpu.VMEM((OB, OUT_DIM), jnp.float32),
            pltpu.VMEM((NHEAD * DEG,), jnp.float32),
            pltpu.SemaphoreType.DMA,
            pltpu.SemaphoreType.DMA,
        ],
    )(_sc_body)
    return f(kv, q, idx2d, mask_p)


def _post_body(hd_ref, q_ref, wfc_ref, bfc_ref, gamma_ref, beta_ref, o_ref):
    dn = (((1,), (1,)), ((), ()))
    x = lax.dot_general(hd_ref[...], wfc_ref[...], dn,
                        preferred_element_type=jnp.float32)
    x = x + bfc_ref[...] + q_ref[...]
    x = 0.5 * x * (1.0 + lax.erf(x * (1.0 / math.sqrt(2.0))))
    mean = jnp.mean(x, axis=1, keepdims=True)
    xc = x - mean
    var = jnp.mean(xc * xc, axis=1, keepdims=True)
    o_ref[...] = xc * lax.rsqrt(var + 1e-5) * gamma_ref[...] + beta_ref[...]


def _postprocess(heads, q, Wfc, bfc, gamma, beta):
    bs = 1000
    grid = (N // bs,)
    return pl.pallas_call(
        _post_body,
        grid=grid,
        in_specs=[
            pl.BlockSpec((bs, OUT_DIM), lambda i: (i, 0)),
            pl.BlockSpec((bs, OUT_DIM), lambda i: (i, 0)),
            pl.BlockSpec((OUT_DIM, OUT_DIM), lambda i: (0, 0)),
            pl.BlockSpec((1, OUT_DIM), lambda i: (0, 0)),
            pl.BlockSpec((1, OUT_DIM), lambda i: (0, 0)),
            pl.BlockSpec((1, OUT_DIM), lambda i: (0, 0)),
        ],
        out_specs=pl.BlockSpec((bs, OUT_DIM), lambda i: (i, 0)),
        out_shape=jax.ShapeDtypeStruct((N, OUT_DIM), jnp.float32),
    )(heads, q, Wfc, bfc.reshape(1, OUT_DIM), gamma.reshape(1, OUT_DIM),
      beta.reshape(1, OUT_DIM))


def kernel(h, neighbor_idx, neighbor_mask, Wq, Wk, Wv, Wfc, bfc, gamma, beta):
    h_p = jnp.pad(h, ((0, NPAD - N), (0, 0)))
    q, kv = _project(h_p, Wq, Wk, Wv)
    idx_p = jnp.pad(neighbor_idx.astype(jnp.int32), ((0, NPAD - N), (0, 0)))
    mask_p = jnp.pad(neighbor_mask, ((0, NPAD - N), (0, 0)), constant_values=1.0)
    idx2d = idx_p.reshape(NPAD * DEG // 128, 128)
    heads = _sc_attention(kv, q, idx2d, mask_p)
    return _postprocess(heads[:N], q[:N], Wfc, bfc, gamma, beta)


# speedup vs baseline: 4.0916x; 2.0636x over previous
import functools
import math

import jax
import jax.numpy as jnp
from jax import lax
from jax.experimental import pallas as pl
from jax.experimental.pallas import tpu as pltpu
from jax.experimental.pallas import tpu_sc as plsc

N = 10000
D = 128
OUT_DIM = 128
NHEAD = 4
HEAD_DIM = 32
DEG = 32

NW = 32
NP_W = 320
NPAD = NW * NP_W
OB = 64
N_OB = NP_W // OB
CH = 4
N_CH = OB // CH

_INV_SQRT_HD = 1.0 / math.sqrt(HEAD_DIM)
KVW = 256


def _proj_body(h_ref, wq_ref, wk_ref, wv_ref, q_ref, kv_ref):
    hb = h_ref[...]
    dn = (((1,), (1,)), ((), ()))
    q = lax.dot_general(hb, wq_ref[...], dn, preferred_element_type=jnp.float32)
    k = lax.dot_general(hb, wk_ref[...], dn, preferred_element_type=jnp.float32)
    v = lax.dot_general(hb, wv_ref[...], dn, preferred_element_type=jnp.float32)
    q_ref[...] = q
    kv_ref[:, 0:OUT_DIM] = k
    kv_ref[:, OUT_DIM:2 * OUT_DIM] = v


def _project(h_p, Wq, Wk, Wv):
    bs = 1024
    grid = (NPAD // bs,)
    return pl.pallas_call(
        _proj_body,
        grid=grid,
        in_specs=[
            pl.BlockSpec((bs, D), lambda i: (i, 0)),
            pl.BlockSpec((OUT_DIM, D), lambda i: (0, 0)),
            pl.BlockSpec((OUT_DIM, D), lambda i: (0, 0)),
            pl.BlockSpec((OUT_DIM, D), lambda i: (0, 0)),
        ],
        out_specs=[
            pl.BlockSpec((bs, OUT_DIM), lambda i: (i, 0)),
            pl.BlockSpec((bs, KVW), lambda i: (i, 0)),
        ],
        out_shape=[
            jax.ShapeDtypeStruct((NPAD, OUT_DIM), jnp.float32),
            jax.ShapeDtypeStruct((NPAD, KVW), jnp.float32),
        ],
    )(h_p, Wq, Wk, Wv)


def _node_compute(kv_buf, q_buf, mask_buf, out_buf, iota16, node, nl):
    rows0 = nl * DEG + iota16
    rows1 = rows0 + 16
    noderow = jnp.full((16,), node, jnp.int32)

    def sbody(c, accs):
        accs = list(accs)
        for u in range(2):
            dimv = (iota16 + (2 * c + u)) & 31
            for h in range(NHEAD):
                colv = dimv + h * HEAD_DIM
                qg = plsc.load_gather(q_buf, [noderow, colv])
                g0 = plsc.load_gather(kv_buf, [rows0, colv])
                g1 = plsc.load_gather(kv_buf, [rows1, colv])
                accs[2 * h] = accs[2 * h] + qg * g0
                accs[2 * h + 1] = accs[2 * h + 1] + qg * g1
        return tuple(accs)

    accs = lax.fori_loop(0, HEAD_DIM // 2, sbody,
                         tuple(jnp.zeros((16,), jnp.float32) for _ in range(8)))

    m0 = mask_buf[node, pl.ds(0, 16)]
    m1 = mask_buf[node, pl.ds(16, 16)]
    es = []
    zinv = []
    for h in range(NHEAD):
        s0 = accs[2 * h] * _INV_SQRT_HD
        s1 = accs[2 * h + 1] * _INV_SQRT_HD
        s0 = jnp.where(m0 == 0.0, -1e9, s0)
        s1 = jnp.where(m1 == 0.0, -1e9, s1)
        mx = jnp.max(jnp.maximum(s0, s1))
        e0 = jnp.exp(s0 - mx)
        e1 = jnp.exp(s1 - mx)
        z = jnp.sum(e0 + e1)
        es.append((e0, e1))
        zinv.append(1.0 / jnp.full((16,), z, jnp.float32))

    def wbody(g, accs):
        av = [jnp.where(g == 0, es[h][0], es[h][1]) for h in range(NHEAD)]
        accs = list(accs)
        for j in range(16):
            row = nl * DEG + g * 16 + j
            for k in range(8):
                vk = kv_buf[row, pl.ds(OUT_DIM + 16 * k, 16)]
                accs[k] = accs[k] + av[k // 2][j] * vk
        return tuple(accs)

    waccs = lax.fori_loop(0, 2, wbody,
                          tuple(jnp.zeros((16,), jnp.float32) for _ in range(8)))
    for k in range(8):
        out_buf[node, pl.ds(16 * k, 16)] = waccs[k] * zinv[k // 2]


def _sc_body(kv_hbm, q_hbm, idx_hbm, mask_hbm, out_hbm,
             kv_buf0, kv_buf1, q_buf, idx_buf, mask_buf, out_buf,
             sem0, sem1):
    cid = lax.axis_index("c")
    sid = lax.axis_index("s")
    wid = sid * 2 + cid
    iota16 = lax.iota(jnp.int32, 16)

    def ob_body(ob, _):
        node0 = wid * NP_W + ob * OB
        pltpu.sync_copy(q_hbm.at[pl.ds(node0, OB)], q_buf)
        pltpu.sync_copy(idx_hbm.at[pl.ds(wid * (NP_W * DEG // 128) + ob * N_CH, N_CH)],
                        idx_buf)
        pltpu.sync_copy(mask_hbm.at[pl.ds(node0, OB)], mask_buf)
        pltpu.async_copy(kv_hbm.at[idx_buf.at[0]], kv_buf0, sem0)

        def cc_body(cc, _):
            pltpu.async_copy(kv_hbm.at[idx_buf.at[2 * cc + 1]], kv_buf1, sem1)
            pltpu.make_async_copy(kv_hbm.at[idx_buf.at[0]], kv_buf0, sem0).wait()

            def n0_body(nl, _):
                _node_compute(kv_buf0, q_buf, mask_buf, out_buf,
                              iota16, (2 * cc) * CH + nl, nl)
                return 0

            lax.fori_loop(0, CH, n0_body, 0)

            @pl.when(cc < N_CH // 2 - 1)
            def _():
                pltpu.async_copy(kv_hbm.at[idx_buf.at[2 * cc + 2]], kv_buf0, sem0)

            pltpu.make_async_copy(kv_hbm.at[idx_buf.at[0]], kv_buf1, sem1).wait()

            def n1_body(nl, _):
                _node_compute(kv_buf1, q_buf, mask_buf, out_buf,
                              iota16, (2 * cc + 1) * CH + nl, nl)
                return 0

            lax.fori_loop(0, CH, n1_body, 0)
            return 0

        lax.fori_loop(0, N_CH // 2, cc_body, 0)
        pltpu.sync_copy(out_buf, out_hbm.at[pl.ds(node0, OB)])
        return 0

    lax.fori_loop(0, N_OB, ob_body, 0)


def _sc_attention(kv, q, idx2d, mask_p):
    mesh = plsc.VectorSubcoreMesh(core_axis_name="c", subcore_axis_name="s")
    f = functools.partial(
        pl.kernel,
        mesh=mesh,
        compiler_params=pltpu.CompilerParams(use_tc_tiling_on_sc=False,
                                             needs_layout_passes=False),
        out_type=jax.ShapeDtypeStruct((NPAD, OUT_DIM), jnp.float32),
        scratch_types=[
            pltpu.VMEM((CH * DEG, KVW), jnp.float32),
            pltpu.VMEM((CH * DEG, KVW), jnp.float32),
            pltpu.VMEM((OB, OUT_DIM), jnp.float32),
            pltpu.VMEM((OB * DEG // 128, 128), jnp.int32),
            pltpu.VMEM((OB, DEG), jnp.float32),
            pltpu.VMEM((OB, OUT_DIM), jnp.float32),
            pltpu.SemaphoreType.DMA,
            pltpu.SemaphoreType.DMA,
        ],
    )(_sc_body)
    return f(kv, q, idx2d, mask_p)


def _post_body(hd_ref, q_ref, wfc_ref, bfc_ref, gamma_ref, beta_ref, o_ref):
    dn = (((1,), (1,)), ((), ()))
    x = lax.dot_general(hd_ref[...], wfc_ref[...], dn,
                        preferred_element_type=jnp.float32)
    x = x + bfc_ref[...] + q_ref[...]
    x = 0.5 * x * (1.0 + lax.erf(x * (1.0 / math.sqrt(2.0))))
    mean = jnp.mean(x, axis=1, keepdims=True)
    xc = x - mean
    var = jnp.mean(xc * xc, axis=1, keepdims=True)
    o_ref[...] = xc * lax.rsqrt(var + 1e-5) * gamma_ref[...] + beta_ref[...]


def _postprocess(heads, q, Wfc, bfc, gamma, beta):
    bs = 1000
    grid = (N // bs,)
    return pl.pallas_call(
        _post_body,
        grid=grid,
        in_specs=[
            pl.BlockSpec((bs, OUT_DIM), lambda i: (i, 0)),
            pl.BlockSpec((bs, OUT_DIM), lambda i: (i, 0)),
            pl.BlockSpec((OUT_DIM, OUT_DIM), lambda i: (0, 0)),
            pl.BlockSpec((1, OUT_DIM), lambda i: (0, 0)),
            pl.BlockSpec((1, OUT_DIM), lambda i: (0, 0)),
            pl.BlockSpec((1, OUT_DIM), lambda i: (0, 0)),
        ],
        out_specs=pl.BlockSpec((bs, OUT_DIM), lambda i: (i, 0)),
        out_shape=jax.ShapeDtypeStruct((N, OUT_DIM), jnp.float32),
    )(heads, q, Wfc, bfc.reshape(1, OUT_DIM), gamma.reshape(1, OUT_DIM),
      beta.reshape(1, OUT_DIM))


def kernel(h, neighbor_idx, neighbor_mask, Wq, Wk, Wv, Wfc, bfc, gamma, beta):
    h_p = jnp.pad(h, ((0, NPAD - N), (0, 0)))
    q, kv = _project(h_p, Wq, Wk, Wv)
    idx_p = jnp.pad(neighbor_idx.astype(jnp.int32), ((0, NPAD - N), (0, 0)))
    mask_p = jnp.pad(neighbor_mask, ((0, NPAD - N), (0, 0)), constant_values=1.0)
    idx2d = idx_p.reshape(NPAD * DEG // 128, 128)
    heads = _sc_attention(kv, q, idx2d, mask_p)
    return _postprocess(heads[:N], q[:N], Wfc, bfc, gamma, beta)
